# baseline (speedup 1.0000x reference)
import functools

import jax
import jax.numpy as jnp
import numpy as np
from jax import lax
from jax.experimental import pallas as pl
from jax.experimental.pallas import tpu as pltpu
from jax.experimental.pallas import tpu_sc as plsc

_N = 10000
_NP = 10240
_E = 320000
_EP = 327680
_EW = _EP // 32
_CH = _EW // 128
_D = 128
_R = 512
_GB = _NP // _R

_PAD_IDX = np.asarray(_N + (np.arange(_EP - _E) % (_NP - _N)), np.int32)

_mesh = plsc.VectorSubcoreMesh(core_axis_name="c", subcore_axis_name="s")


@functools.partial(
    pl.kernel,
    out_type=jax.ShapeDtypeStruct((2, _NP), jnp.float32),
    mesh=_mesh,
    scratch_types=[
        pltpu.VMEM((_CH, 128), jnp.int32),
        pltpu.VMEM((128,), jnp.float32),
        pltpu.VMEM((128,), jnp.float32),
        pltpu.VMEM_SHARED((_NP,), jnp.float32),
        pltpu.SemaphoreType.DMA,
    ],
)
def _deg_kernel(dst_hbm, out_hbm, di_all, ones_v, zero_v, deg_sp, sem):
    cid = lax.axis_index("c")
    sid = lax.axis_index("s")
    w = cid * 16 + sid
    for j in range(8):
        ones_v[pl.ds(j * 16, 16)] = jnp.full((16,), 1.0, jnp.float32)
        zero_v[pl.ds(j * 16, 16)] = jnp.zeros((16,), jnp.float32)
    pltpu.sync_copy(dst_hbm.at[pl.ds(w * _CH, _CH), :], di_all)
    for j in range(5):
        pltpu.sync_copy(zero_v, deg_sp.at[pl.ds(sid * 640 + j * 128, 128)])
    plsc.subcore_barrier()

    def group(g, carry):
        def fire(k, c):
            pltpu.async_copy(ones_v, deg_sp.at[di_all.at[g * 16 + k]], sem, add=True)
            return c

        lax.fori_loop(0, 16, fire, 0)

        def drain(k, c):
            pltpu.make_async_copy(ones_v, deg_sp.at[di_all.at[g * 16 + k]], sem).wait()
            return c

        lax.fori_loop(0, 16, drain, 0)
        return carry

    lax.fori_loop(0, _CH // 16, group, 0)
    plsc.subcore_barrier()
    for j in range(5):
        r0 = sid * 640 + j * 128
        pltpu.sync_copy(deg_sp.at[pl.ds(r0, 128)], out_hbm.at[cid, pl.ds(r0, 128)])


def _mm_body(x_ref, w_ref, dv_ref, u_ref):
    h = jnp.dot(x_ref[...], w_ref[...], preferred_element_type=jnp.float32)
    dv = dv_ref[0]
    eye = (jax.lax.broadcasted_iota(jnp.int32, (128, 128), 0)
           == jax.lax.broadcasted_iota(jnp.int32, (128, 128), 1))
    for g in range(_R // 128):
        dg = jnp.where(eye, jnp.broadcast_to(dv[g:g + 1, :], (128, 128)), 0.0)
        u_ref[pl.ds(g * 128, 128), :] = jnp.dot(
            dg, h[g * 128:(g + 1) * 128, :], preferred_element_type=jnp.float32)


_mm = pl.pallas_call(
    _mm_body,
    grid=(_GB,),
    in_specs=[
        pl.BlockSpec((_R, _D), lambda i: (i, 0)),
        pl.BlockSpec((_D, _D), lambda i: (0, 0)),
        pl.BlockSpec((1, _R // 128, 128), lambda i: (i, 0, 0)),
    ],
    out_specs=pl.BlockSpec((_R, _D), lambda i: (i, 0)),
    out_shape=jax.ShapeDtypeStruct((_NP, _D), jnp.float32),
)


@functools.partial(
    pl.kernel,
    out_type=(
        jax.ShapeDtypeStruct((2, _NP, _D), jnp.float32),
        jax.ShapeDtypeStruct((2, _NP), jnp.float32),
    ),
    mesh=_mesh,
    scratch_types=[
        pltpu.VMEM((_CH // 5, 128), jnp.int32),
        pltpu.VMEM((_CH // 5, 128), jnp.int32),
        pltpu.VMEM((128, _D), jnp.float32),
        pltpu.VMEM((128, _D), jnp.float32),
        pltpu.VMEM((128,), jnp.float32),
        pltpu.VMEM_SHARED((_NP, _D), jnp.float32),
        pltpu.VMEM_SHARED((_NP,), jnp.float32),
        pltpu.SemaphoreType.DMA,
        pltpu.SemaphoreType.DMA,
    ],
)
def _agg_kernel(u_hbm, src_hbm, dst_hbm, dinv_hbm, agg_out, s_out,
                si_q, di_q, rows0, rows1, vals_v, agg_sp, s_sp, gu0, gu1):
    cid = lax.axis_index("c")
    sid = lax.axis_index("s")
    w = cid * 16 + sid
    pc = _CH // 5

    def zb(i, carry):
        r = i // 8
        c = (i % 8) * 16
        rows0[r, pl.ds(c, 16)] = jnp.zeros((16,), jnp.float32)
        return carry

    lax.fori_loop(0, 1024, zb, 0)
    for j in range(8):
        vals_v[pl.ds(j * 16, 16)] = jnp.zeros((16,), jnp.float32)
    for j in range(5):
        r0 = sid * 640 + j * 128
        pltpu.sync_copy(rows0, agg_sp.at[pl.ds(r0, 128), :])
        pltpu.sync_copy(vals_v, s_sp.at[pl.ds(r0, 128)])
    plsc.subcore_barrier()

    def phase(p, carry):
        pltpu.sync_copy(src_hbm.at[pl.ds(w * _CH + p * pc, pc), :], si_q)
        pltpu.sync_copy(dst_hbm.at[pl.ds(w * _CH + p * pc, pc), :], di_q)
        pltpu.async_copy(u_hbm.at[si_q.at[0]], rows0, gu0)
        pltpu.async_copy(u_hbm.at[si_q.at[1]], rows1, gu1)

        def schunk(c, cc):
            pltpu.sync_copy(dinv_hbm.at[di_q.at[c]], vals_v)
            pltpu.sync_copy(vals_v, s_sp.at[si_q.at[c]], add=True)
            return cc

        lax.fori_loop(0, pc, schunk, 0)

        def body(i, c):
            c0 = i * 2
            c1 = i * 2 + 1
            pltpu.make_async_copy(u_hbm.at[si_q.at[c0]], rows0, gu0).wait()
            pltpu.sync_copy(rows0, agg_sp.at[di_q.at[c0]], add=True)

            @pl.when(c0 + 2 < pc)
            def _():
                pltpu.async_copy(u_hbm.at[si_q.at[c0 + 2]], rows0, gu0)

            pltpu.make_async_copy(u_hbm.at[si_q.at[c1]], rows1, gu1).wait()
            pltpu.sync_copy(rows1, agg_sp.at[di_q.at[c1]], add=True)

            @pl.when(c1 + 2 < pc)
            def _():
                pltpu.async_copy(u_hbm.at[si_q.at[c1 + 2]], rows1, gu1)

            return c

        lax.fori_loop(0, pc // 2, body, 0)
        return carry

    lax.fori_loop(0, 5, phase, 0)
    plsc.subcore_barrier()
    for j in range(5):
        r0 = sid * 640 + j * 128
        pltpu.sync_copy(agg_sp.at[pl.ds(r0, 128), :], agg_out.at[cid, pl.ds(r0, 128), :])
        pltpu.sync_copy(s_sp.at[pl.ds(r0, 128)], s_out.at[cid, pl.ds(r0, 128)])


def _fin_body(u_ref, agg_ref, dv_ref, s_ref, b1_ref, w2_ref, b2_ref,
              out_ref, acc_ref):
    i = pl.program_id(0)

    @pl.when(i == 0)
    def _():
        acc_ref[...] = jnp.zeros_like(acc_ref)

    dv44 = dv_ref[0]
    s44 = s_ref[0]
    w44 = dv44 * (s44 + dv44)
    gidx = (i * _R + 128 * jax.lax.broadcasted_iota(jnp.int32, (_R // 128, 128), 0)
            + jax.lax.broadcasted_iota(jnp.int32, (_R // 128, 128), 1))
    w44 = jnp.where(gidx < _N, w44, 0.0)
    eye = (jax.lax.broadcasted_iota(jnp.int32, (128, 128), 0)
           == jax.lax.broadcasted_iota(jnp.int32, (128, 128), 1))
    m = agg_ref[0] + agg_ref[1] + u_ref[...]
    part = jnp.zeros((1, _D), jnp.float32)
    for g in range(_R // 128):
        dg = jnp.where(eye, jnp.broadcast_to(dv44[g:g + 1, :], (128, 128)), 0.0)
        h1g = jnp.maximum(
            jnp.dot(dg, m[g * 128:(g + 1) * 128, :],
                    preferred_element_type=jnp.float32) + b1_ref[...], 0.0)
        part += jnp.dot(w44[g:g + 1, :], h1g, preferred_element_type=jnp.float32)
    acc_ref[...] += part

    @pl.when(i == _GB - 1)
    def _():
        v = jnp.dot(acc_ref[...], w2_ref[...], preferred_element_type=jnp.float32)
        out_ref[...] = jnp.tanh(v * (1.0 / _N) + b2_ref[...])


_fin = pl.pallas_call(
    _fin_body,
    grid=(_GB,),
    in_specs=[
        pl.BlockSpec((_R, _D), lambda i: (i, 0)),
        pl.BlockSpec((2, _R, _D), lambda i: (0, i, 0)),
        pl.BlockSpec((1, _R // 128, 128), lambda i: (i, 0, 0)),
        pl.BlockSpec((1, _R // 128, 128), lambda i: (i, 0, 0)),
        pl.BlockSpec((1, _D), lambda i: (0, 0)),
        pl.BlockSpec((_D, _D), lambda i: (0, 0)),
        pl.BlockSpec((1, _D), lambda i: (0, 0)),
    ],
    out_specs=pl.BlockSpec((1, _D), lambda i: (0, 0)),
    out_shape=jax.ShapeDtypeStruct((1, _D), jnp.float32),
    scratch_shapes=[pltpu.VMEM((1, _D), jnp.float32)],
)


def kernel(x, edge_index, W1, b1, W2, b2):
    pad2d = jnp.asarray(_PAD_IDX.reshape(-1, 128))
    src = jnp.concatenate([edge_index[0].reshape(_E // 128, 128), pad2d])
    dst = jnp.concatenate([edge_index[1].reshape(_E // 128, 128), pad2d])
    x_pad = jnp.pad(x, ((0, _NP - _N), (0, 0)))

    deg2 = _deg_kernel(dst)
    dinv = lax.rsqrt(deg2[0] + deg2[1] + 1.0)
    dinv_flat = dinv.reshape(_GB, _R // 128, 128)

    u_pad = _mm(x_pad, W1, dinv_flat)
    agg2, s2 = _agg_kernel(u_pad, src, dst, dinv)

    s_flat = (s2[0] + s2[1]).reshape(_GB, _R // 128, 128)

    return _fin(u_pad, agg2, dinv_flat, s_flat,
                b1.reshape(1, _D), W2, b2.reshape(1, _D))


# speedup vs baseline: 1.0572x; 1.0572x over previous
import functools

import jax
import jax.numpy as jnp
import numpy as np
from jax import lax
from jax.experimental import pallas as pl
from jax.experimental.pallas import tpu as pltpu
from jax.experimental.pallas import tpu_sc as plsc

_N = 10000
_NP = 10240
_E = 320000
_EP = 327680
_EW = _EP // 32
_CH = _EW // 128
_D = 128
_R = 512
_GB = _NP // _R

_PAD_IDX = np.asarray(_N + (np.arange(_EP - _E) % (_NP - _N)), np.int32)

_mesh = plsc.VectorSubcoreMesh(core_axis_name="c", subcore_axis_name="s")


@functools.partial(
    pl.kernel,
    out_type=jax.ShapeDtypeStruct((2, _NP), jnp.float32),
    mesh=_mesh,
    scratch_types=[
        pltpu.VMEM((_CH, 128), jnp.int32),
        pltpu.VMEM((128,), jnp.float32),
        pltpu.VMEM((128,), jnp.float32),
        pltpu.VMEM_SHARED((_NP,), jnp.float32),
        pltpu.SemaphoreType.DMA,
    ],
)
def _deg_kernel(dst_hbm, out_hbm, di_all, ones_v, zero_v, deg_sp, sem):
    cid = lax.axis_index("c")
    sid = lax.axis_index("s")
    w = cid * 16 + sid
    for j in range(8):
        ones_v[pl.ds(j * 16, 16)] = jnp.full((16,), 1.0, jnp.float32)
        zero_v[pl.ds(j * 16, 16)] = jnp.zeros((16,), jnp.float32)
    pltpu.sync_copy(dst_hbm.at[pl.ds(w * _CH, _CH), :], di_all)
    for j in range(5):
        pltpu.sync_copy(zero_v, deg_sp.at[pl.ds(sid * 640 + j * 128, 128)])
    plsc.subcore_barrier()

    def group(g, carry):
        def fire(k, c):
            pltpu.async_copy(ones_v, deg_sp.at[di_all.at[g * 16 + k]], sem, add=True)
            return c

        lax.fori_loop(0, 16, fire, 0)

        def drain(k, c):
            pltpu.make_async_copy(ones_v, deg_sp.at[di_all.at[g * 16 + k]], sem).wait()
            return c

        lax.fori_loop(0, 16, drain, 0)
        return carry

    lax.fori_loop(0, _CH // 16, group, 0)
    plsc.subcore_barrier()
    for j in range(5):
        r0 = sid * 640 + j * 128
        pltpu.sync_copy(deg_sp.at[pl.ds(r0, 128)], out_hbm.at[cid, pl.ds(r0, 128)])


def _mm_body(x_ref, w_ref, dv_ref, u_ref):
    h = jnp.dot(x_ref[...], w_ref[...], preferred_element_type=jnp.float32)
    dv = dv_ref[0]
    eye = (jax.lax.broadcasted_iota(jnp.int32, (128, 128), 0)
           == jax.lax.broadcasted_iota(jnp.int32, (128, 128), 1))
    for g in range(_R // 128):
        dg = jnp.where(eye, jnp.broadcast_to(dv[g:g + 1, :], (128, 128)), 0.0)
        u_ref[pl.ds(g * 128, 128), :] = jnp.dot(
            dg, h[g * 128:(g + 1) * 128, :], preferred_element_type=jnp.float32)


_mm = pl.pallas_call(
    _mm_body,
    grid=(_GB,),
    in_specs=[
        pl.BlockSpec((_R, _D), lambda i: (i, 0)),
        pl.BlockSpec((_D, _D), lambda i: (0, 0)),
        pl.BlockSpec((1, _R // 128, 128), lambda i: (i, 0, 0)),
    ],
    out_specs=pl.BlockSpec((_R, _D), lambda i: (i, 0)),
    out_shape=jax.ShapeDtypeStruct((_NP, _D), jnp.float32),
)


@functools.partial(
    pl.kernel,
    out_type=(
        jax.ShapeDtypeStruct((2, _NP, _D), jnp.float32),
        jax.ShapeDtypeStruct((2, _NP), jnp.float32),
    ),
    mesh=_mesh,
    scratch_types=[
        pltpu.VMEM((_CH // 5, 128), jnp.int32),
        pltpu.VMEM((_CH // 5, 128), jnp.int32),
        pltpu.VMEM((128, _D), jnp.float32),
        pltpu.VMEM((128, _D), jnp.float32),
        pltpu.VMEM((128,), jnp.float32),
        pltpu.VMEM_SHARED((_NP, _D), jnp.float32),
        pltpu.VMEM_SHARED((_NP,), jnp.float32),
        pltpu.SemaphoreType.DMA,
        pltpu.SemaphoreType.DMA,
    ],
)
def _agg_kernel(u_hbm, src_hbm, dst_hbm, dinv_hbm, agg_out, s_out,
                si_q, di_q, rows0, rows1, vals_v, agg_sp, s_sp, gu0, gu1):
    cid = lax.axis_index("c")
    sid = lax.axis_index("s")
    w = cid * 16 + sid
    pc = _CH // 5

    def zb(i, carry):
        r = i // 8
        c = (i % 8) * 16
        rows0[r, pl.ds(c, 16)] = jnp.zeros((16,), jnp.float32)
        return carry

    lax.fori_loop(0, 1024, zb, 0)
    for j in range(8):
        vals_v[pl.ds(j * 16, 16)] = jnp.zeros((16,), jnp.float32)
    for j in range(5):
        r0 = sid * 640 + j * 128
        pltpu.sync_copy(rows0, agg_sp.at[pl.ds(r0, 128), :])
        pltpu.sync_copy(vals_v, s_sp.at[pl.ds(r0, 128)])
    plsc.subcore_barrier()

    def phase(p, carry):
        pltpu.sync_copy(src_hbm.at[pl.ds(w * _CH + p * pc, pc), :], si_q)
        pltpu.sync_copy(dst_hbm.at[pl.ds(w * _CH + p * pc, pc), :], di_q)
        pltpu.async_copy(u_hbm.at[si_q.at[0]], rows0, gu0)
        pltpu.async_copy(u_hbm.at[si_q.at[1]], rows1, gu1)

        def body(i, c):
            c0 = i * 2
            c1 = i * 2 + 1
            pltpu.sync_copy(dinv_hbm.at[di_q.at[c0]], vals_v)
            pltpu.sync_copy(vals_v, s_sp.at[si_q.at[c0]], add=True)
            pltpu.make_async_copy(u_hbm.at[si_q.at[c0]], rows0, gu0).wait()
            pltpu.sync_copy(rows0, agg_sp.at[di_q.at[c0]], add=True)

            @pl.when(c0 + 2 < pc)
            def _():
                pltpu.async_copy(u_hbm.at[si_q.at[c0 + 2]], rows0, gu0)

            pltpu.sync_copy(dinv_hbm.at[di_q.at[c1]], vals_v)
            pltpu.sync_copy(vals_v, s_sp.at[si_q.at[c1]], add=True)
            pltpu.make_async_copy(u_hbm.at[si_q.at[c1]], rows1, gu1).wait()
            pltpu.sync_copy(rows1, agg_sp.at[di_q.at[c1]], add=True)

            @pl.when(c1 + 2 < pc)
            def _():
                pltpu.async_copy(u_hbm.at[si_q.at[c1 + 2]], rows1, gu1)

            return c

        lax.fori_loop(0, pc // 2, body, 0)
        return carry

    lax.fori_loop(0, 5, phase, 0)
    plsc.subcore_barrier()
    for j in range(5):
        r0 = sid * 640 + j * 128
        pltpu.sync_copy(agg_sp.at[pl.ds(r0, 128), :], agg_out.at[cid, pl.ds(r0, 128), :])
        pltpu.sync_copy(s_sp.at[pl.ds(r0, 128)], s_out.at[cid, pl.ds(r0, 128)])


def _fin_body(u_ref, agg_ref, dv_ref, s_ref, b1_ref, w2_ref, b2_ref,
              out_ref, acc_ref):
    i = pl.program_id(0)

    @pl.when(i == 0)
    def _():
        acc_ref[...] = jnp.zeros_like(acc_ref)

    dv44 = dv_ref[0]
    s44 = s_ref[0]
    w44 = dv44 * (s44 + dv44)
    gidx = (i * _R + 128 * jax.lax.broadcasted_iota(jnp.int32, (_R // 128, 128), 0)
            + jax.lax.broadcasted_iota(jnp.int32, (_R // 128, 128), 1))
    w44 = jnp.where(gidx < _N, w44, 0.0)
    eye = (jax.lax.broadcasted_iota(jnp.int32, (128, 128), 0)
           == jax.lax.broadcasted_iota(jnp.int32, (128, 128), 1))
    m = agg_ref[0] + agg_ref[1] + u_ref[...]
    part = jnp.zeros((1, _D), jnp.float32)
    for g in range(_R // 128):
        dg = jnp.where(eye, jnp.broadcast_to(dv44[g:g + 1, :], (128, 128)), 0.0)
        h1g = jnp.maximum(
            jnp.dot(dg, m[g * 128:(g + 1) * 128, :],
                    preferred_element_type=jnp.float32) + b1_ref[...], 0.0)
        part += jnp.dot(w44[g:g + 1, :], h1g, preferred_element_type=jnp.float32)
    acc_ref[...] += part

    @pl.when(i == _GB - 1)
    def _():
        v = jnp.dot(acc_ref[...], w2_ref[...], preferred_element_type=jnp.float32)
        out_ref[...] = jnp.tanh(v * (1.0 / _N) + b2_ref[...])


_fin = pl.pallas_call(
    _fin_body,
    grid=(_GB,),
    in_specs=[
        pl.BlockSpec((_R, _D), lambda i: (i, 0)),
        pl.BlockSpec((2, _R, _D), lambda i: (0, i, 0)),
        pl.BlockSpec((1, _R // 128, 128), lambda i: (i, 0, 0)),
        pl.BlockSpec((1, _R // 128, 128), lambda i: (i, 0, 0)),
        pl.BlockSpec((1, _D), lambda i: (0, 0)),
        pl.BlockSpec((_D, _D), lambda i: (0, 0)),
        pl.BlockSpec((1, _D), lambda i: (0, 0)),
    ],
    out_specs=pl.BlockSpec((1, _D), lambda i: (0, 0)),
    out_shape=jax.ShapeDtypeStruct((1, _D), jnp.float32),
    scratch_shapes=[pltpu.VMEM((1, _D), jnp.float32)],
)


def kernel(x, edge_index, W1, b1, W2, b2):
    pad2d = jnp.asarray(_PAD_IDX.reshape(-1, 128))
    src = jnp.concatenate([edge_index[0].reshape(_E // 128, 128), pad2d])
    dst = jnp.concatenate([edge_index[1].reshape(_E // 128, 128), pad2d])
    x_pad = jnp.pad(x, ((0, _NP - _N), (0, 0)))

    deg2 = _deg_kernel(dst)
    dinv = lax.rsqrt(deg2[0] + deg2[1] + 1.0)
    dinv_flat = dinv.reshape(_GB, _R // 128, 128)

    u_pad = _mm(x_pad, W1, dinv_flat)
    agg2, s2 = _agg_kernel(u_pad, src, dst, dinv)

    s_flat = (s2[0] + s2[1]).reshape(_GB, _R // 128, 128)

    return _fin(u_pad, agg2, dinv_flat, s_flat,
                b1.reshape(1, _D), W2, b2.reshape(1, _D))


# speedup vs baseline: 1.3117x; 1.2407x over previous
import functools

import jax
import jax.numpy as jnp
import numpy as np
from jax import lax
from jax.experimental import pallas as pl
from jax.experimental.pallas import tpu as pltpu
from jax.experimental.pallas import tpu_sc as plsc

_N = 10000
_NP = 10240
_E = 320000
_EP = 327680
_EW = _EP // 32
_CH = _EW // 128
_D = 128
_R = 512
_GB = _NP // _R

_PAD_IDX = np.asarray(_N + (np.arange(_EP - _E) % (_NP - _N)), np.int32)

_mesh = plsc.VectorSubcoreMesh(core_axis_name="c", subcore_axis_name="s")


@functools.partial(
    pl.kernel,
    out_type=jax.ShapeDtypeStruct((2, _NP), jnp.float32),
    mesh=_mesh,
    scratch_types=[
        pltpu.VMEM((_CH, 128), jnp.int32),
        pltpu.VMEM((128,), jnp.float32),
        pltpu.VMEM((128,), jnp.float32),
        pltpu.VMEM_SHARED((_NP,), jnp.float32),
        pltpu.SemaphoreType.DMA,
    ],
)
def _deg_kernel(dst_hbm, out_hbm, di_all, ones_v, zero_v, deg_sp, sem):
    cid = lax.axis_index("c")
    sid = lax.axis_index("s")
    w = cid * 16 + sid
    for j in range(8):
        ones_v[pl.ds(j * 16, 16)] = jnp.full((16,), 1.0, jnp.float32)
        zero_v[pl.ds(j * 16, 16)] = jnp.zeros((16,), jnp.float32)
    pltpu.sync_copy(dst_hbm.at[pl.ds(w * _CH, _CH), :], di_all)
    for j in range(5):
        pltpu.sync_copy(zero_v, deg_sp.at[pl.ds(sid * 640 + j * 128, 128)])
    plsc.subcore_barrier()

    def group(g, carry):
        def fire(k, c):
            pltpu.async_copy(ones_v, deg_sp.at[di_all.at[g * 16 + k]], sem, add=True)
            return c

        lax.fori_loop(0, 16, fire, 0)

        def drain(k, c):
            pltpu.make_async_copy(ones_v, deg_sp.at[di_all.at[g * 16 + k]], sem).wait()
            return c

        lax.fori_loop(0, 16, drain, 0)
        return carry

    lax.fori_loop(0, _CH // 16, group, 0)
    plsc.subcore_barrier()
    for j in range(5):
        r0 = sid * 640 + j * 128
        pltpu.sync_copy(deg_sp.at[pl.ds(r0, 128)], out_hbm.at[cid, pl.ds(r0, 128)])


def _mm_body(x_ref, w_ref, dv_ref, u_ref):
    h = jnp.dot(x_ref[...], w_ref[...], preferred_element_type=jnp.float32)
    dv = dv_ref[0]
    eye = (jax.lax.broadcasted_iota(jnp.int32, (128, 128), 0)
           == jax.lax.broadcasted_iota(jnp.int32, (128, 128), 1))
    for g in range(_R // 128):
        dg = jnp.where(eye, jnp.broadcast_to(dv[g:g + 1, :], (128, 128)), 0.0)
        u_ref[pl.ds(g * 128, 128), :] = jnp.dot(
            dg, h[g * 128:(g + 1) * 128, :], preferred_element_type=jnp.float32)


_mm = pl.pallas_call(
    _mm_body,
    grid=(_GB,),
    in_specs=[
        pl.BlockSpec((_R, _D), lambda i: (i, 0)),
        pl.BlockSpec((_D, _D), lambda i: (0, 0)),
        pl.BlockSpec((1, _R // 128, 128), lambda i: (i, 0, 0)),
    ],
    out_specs=pl.BlockSpec((_R, _D), lambda i: (i, 0)),
    out_shape=jax.ShapeDtypeStruct((_NP, _D), jnp.float32),
)


@functools.partial(
    pl.kernel,
    out_type=(
        jax.ShapeDtypeStruct((2, _NP, _D), jnp.float32),
        jax.ShapeDtypeStruct((2, _NP), jnp.float32),
    ),
    mesh=_mesh,
    scratch_types=[
        pltpu.VMEM((_CH // 5, 128), jnp.int32),
        pltpu.VMEM((_CH // 5, 128), jnp.int32),
        pltpu.VMEM((128, _D), jnp.float32),
        pltpu.VMEM((128, _D), jnp.float32),
        pltpu.VMEM((128,), jnp.float32),
        pltpu.VMEM_SHARED((_NP, _D), jnp.float32),
        pltpu.VMEM_SHARED((_NP,), jnp.float32),
        pltpu.VMEM_SHARED((_NP,), jnp.float32),
        pltpu.SemaphoreType.DMA,
        pltpu.SemaphoreType.DMA,
    ],
)
def _agg_kernel(u_hbm, src_hbm, dst_hbm, dinv_hbm, agg_out, s_out,
                si_q, di_q, rows0, rows1, vals_v, agg_sp, s_sp, dinv_sh,
                gu0, gu1):
    cid = lax.axis_index("c")
    sid = lax.axis_index("s")
    w = cid * 16 + sid
    pc = _CH // 5

    def zb(i, carry):
        r = i // 8
        c = (i % 8) * 16
        rows0[r, pl.ds(c, 16)] = jnp.zeros((16,), jnp.float32)
        return carry

    lax.fori_loop(0, 1024, zb, 0)
    for j in range(8):
        vals_v[pl.ds(j * 16, 16)] = jnp.zeros((16,), jnp.float32)
    for j in range(5):
        r0 = sid * 640 + j * 128
        pltpu.sync_copy(rows0, agg_sp.at[pl.ds(r0, 128), :])
        pltpu.sync_copy(vals_v, s_sp.at[pl.ds(r0, 128)])
        pltpu.sync_copy(dinv_hbm.at[pl.ds(r0, 128)], dinv_sh.at[pl.ds(r0, 128)])
    plsc.subcore_barrier()

    def phase(p, carry):
        pltpu.sync_copy(src_hbm.at[pl.ds(w * _CH + p * pc, pc), :], si_q)
        pltpu.sync_copy(dst_hbm.at[pl.ds(w * _CH + p * pc, pc), :], di_q)
        pltpu.async_copy(u_hbm.at[si_q.at[0]], rows0, gu0)
        pltpu.async_copy(u_hbm.at[si_q.at[1]], rows1, gu1)

        def body(i, c):
            c0 = i * 2
            c1 = i * 2 + 1
            pltpu.sync_copy(dinv_sh.at[di_q.at[c0]], vals_v)
            pltpu.sync_copy(vals_v, s_sp.at[si_q.at[c0]], add=True)
            pltpu.make_async_copy(u_hbm.at[si_q.at[c0]], rows0, gu0).wait()
            pltpu.sync_copy(rows0, agg_sp.at[di_q.at[c0]], add=True)

            @pl.when(c0 + 2 < pc)
            def _():
                pltpu.async_copy(u_hbm.at[si_q.at[c0 + 2]], rows0, gu0)

            pltpu.sync_copy(dinv_sh.at[di_q.at[c1]], vals_v)
            pltpu.sync_copy(vals_v, s_sp.at[si_q.at[c1]], add=True)
            pltpu.make_async_copy(u_hbm.at[si_q.at[c1]], rows1, gu1).wait()
            pltpu.sync_copy(rows1, agg_sp.at[di_q.at[c1]], add=True)

            @pl.when(c1 + 2 < pc)
            def _():
                pltpu.async_copy(u_hbm.at[si_q.at[c1 + 2]], rows1, gu1)

            return c

        lax.fori_loop(0, pc // 2, body, 0)
        return carry

    lax.fori_loop(0, 5, phase, 0)
    plsc.subcore_barrier()
    for j in range(5):
        r0 = sid * 640 + j * 128
        pltpu.sync_copy(agg_sp.at[pl.ds(r0, 128), :], agg_out.at[cid, pl.ds(r0, 128), :])
        pltpu.sync_copy(s_sp.at[pl.ds(r0, 128)], s_out.at[cid, pl.ds(r0, 128)])


def _fin_body(u_ref, agg_ref, dv_ref, s_ref, b1_ref, w2_ref, b2_ref,
              out_ref, acc_ref):
    i = pl.program_id(0)

    @pl.when(i == 0)
    def _():
        acc_ref[...] = jnp.zeros_like(acc_ref)

    dv44 = dv_ref[0]
    s44 = s_ref[0]
    w44 = dv44 * (s44 + dv44)
    gidx = (i * _R + 128 * jax.lax.broadcasted_iota(jnp.int32, (_R // 128, 128), 0)
            + jax.lax.broadcasted_iota(jnp.int32, (_R // 128, 128), 1))
    w44 = jnp.where(gidx < _N, w44, 0.0)
    eye = (jax.lax.broadcasted_iota(jnp.int32, (128, 128), 0)
           == jax.lax.broadcasted_iota(jnp.int32, (128, 128), 1))
    m = agg_ref[0] + agg_ref[1] + u_ref[...]
    part = jnp.zeros((1, _D), jnp.float32)
    for g in range(_R // 128):
        dg = jnp.where(eye, jnp.broadcast_to(dv44[g:g + 1, :], (128, 128)), 0.0)
        h1g = jnp.maximum(
            jnp.dot(dg, m[g * 128:(g + 1) * 128, :],
                    preferred_element_type=jnp.float32) + b1_ref[...], 0.0)
        part += jnp.dot(w44[g:g + 1, :], h1g, preferred_element_type=jnp.float32)
    acc_ref[...] += part

    @pl.when(i == _GB - 1)
    def _():
        v = jnp.dot(acc_ref[...], w2_ref[...], preferred_element_type=jnp.float32)
        out_ref[...] = jnp.tanh(v * (1.0 / _N) + b2_ref[...])


_fin = pl.pallas_call(
    _fin_body,
    grid=(_GB,),
    in_specs=[
        pl.BlockSpec((_R, _D), lambda i: (i, 0)),
        pl.BlockSpec((2, _R, _D), lambda i: (0, i, 0)),
        pl.BlockSpec((1, _R // 128, 128), lambda i: (i, 0, 0)),
        pl.BlockSpec((1, _R // 128, 128), lambda i: (i, 0, 0)),
        pl.BlockSpec((1, _D), lambda i: (0, 0)),
        pl.BlockSpec((_D, _D), lambda i: (0, 0)),
        pl.BlockSpec((1, _D), lambda i: (0, 0)),
    ],
    out_specs=pl.BlockSpec((1, _D), lambda i: (0, 0)),
    out_shape=jax.ShapeDtypeStruct((1, _D), jnp.float32),
    scratch_shapes=[pltpu.VMEM((1, _D), jnp.float32)],
)


def kernel(x, edge_index, W1, b1, W2, b2):
    pad2d = jnp.asarray(_PAD_IDX.reshape(-1, 128))
    src = jnp.concatenate([edge_index[0].reshape(_E // 128, 128), pad2d])
    dst = jnp.concatenate([edge_index[1].reshape(_E // 128, 128), pad2d])
    x_pad = jnp.pad(x, ((0, _NP - _N), (0, 0)))

    deg2 = _deg_kernel(dst)
    dinv = lax.rsqrt(deg2[0] + deg2[1] + 1.0)
    dinv_flat = dinv.reshape(_GB, _R // 128, 128)

    u_pad = _mm(x_pad, W1, dinv_flat)
    agg2, s2 = _agg_kernel(u_pad, src, dst, dinv)

    s_flat = (s2[0] + s2[1]).reshape(_GB, _R // 128, 128)

    return _fin(u_pad, agg2, dinv_flat, s_flat,
                b1.reshape(1, _D), W2, b2.reshape(1, _D))
